# baseline (speedup 1.0000x reference)
import jax
import jax.numpy as jnp
from jax.experimental import pallas as pl
from jax.experimental.pallas import tpu as pltpu

_NCHUNK = 4


def _dma_copy(emb_ref, out_ref, sems):
    T = out_ref.shape[1]
    C = T // _NCHUNK
    for i in range(_NCHUNK):
        pltpu.make_async_copy(
            emb_ref.at[pl.ds(i * C, C), :],
            out_ref.at[0, pl.ds(i * C, C), :],
            sems.at[i],
        ).start()
    for i in range(_NCHUNK):
        pltpu.make_async_copy(
            emb_ref.at[pl.ds(i * C, C), :],
            out_ref.at[0, pl.ds(i * C, C), :],
            sems.at[i],
        ).wait()


def kernel(x, pos_emb):
    T = x.shape[1]
    D = pos_emb.shape[1]
    out = pl.pallas_call(
        _dma_copy,
        in_specs=[pl.BlockSpec(memory_space=pltpu.MemorySpace.HBM)],
        out_specs=pl.BlockSpec(memory_space=pltpu.MemorySpace.HBM),
        out_shape=jax.ShapeDtypeStruct((1, T, D), pos_emb.dtype),
        scratch_shapes=[pltpu.SemaphoreType.DMA((_NCHUNK,))],
    )(pos_emb)
    return out


# speedup vs baseline: 40.9610x; 40.9610x over previous
import jax
import jax.numpy as jnp
from jax.experimental import pallas as pl


def _copy_block(emb_ref, out_ref):
    out_ref[0, :, :] = emb_ref[:, :]


def kernel(x, pos_emb):
    T = x.shape[1]
    D = pos_emb.shape[1]
    R = 4096
    out = pl.pallas_call(
        _copy_block,
        grid=(T // R,),
        in_specs=[pl.BlockSpec((R, D), lambda i: (i, 0))],
        out_specs=pl.BlockSpec((1, R, D), lambda i: (0, i, 0)),
        out_shape=jax.ShapeDtypeStruct((1, T, D), pos_emb.dtype),
    )(pos_emb)
    return out


# speedup vs baseline: 45.7828x; 1.1177x over previous
import jax
import jax.numpy as jnp
from jax.experimental import pallas as pl
from jax.experimental.pallas import tpu as pltpu

_NCHUNK = 4


def _dma_copy(emb_ref, out_ref, scratch, sems):
    T = out_ref.shape[1]
    C = T // _NCHUNK

    def in_copy(i):
        return pltpu.make_async_copy(
            emb_ref.at[pl.ds(i * C, C), :],
            scratch.at[pl.ds(i * C, C), :],
            sems.at[i],
        )

    def out_copy(i):
        return pltpu.make_async_copy(
            scratch.at[pl.ds(i * C, C), :],
            out_ref.at[0, pl.ds(i * C, C), :],
            sems.at[_NCHUNK + i],
        )

    for i in range(_NCHUNK):
        in_copy(i).start()
    for i in range(_NCHUNK):
        in_copy(i).wait()
        out_copy(i).start()
    for i in range(_NCHUNK):
        out_copy(i).wait()


def kernel(x, pos_emb):
    T = x.shape[1]
    D = pos_emb.shape[1]
    out = pl.pallas_call(
        _dma_copy,
        in_specs=[pl.BlockSpec(memory_space=pltpu.MemorySpace.HBM)],
        out_specs=pl.BlockSpec(memory_space=pltpu.MemorySpace.HBM),
        out_shape=jax.ShapeDtypeStruct((1, T, D), pos_emb.dtype),
        scratch_shapes=[
            pltpu.VMEM((T, D), pos_emb.dtype),
            pltpu.SemaphoreType.DMA((2 * _NCHUNK,)),
        ],
    )(pos_emb)
    return out
